# Initial kernel scaffold; baseline (speedup 1.0000x reference)
import functools

import jax
import jax.numpy as jnp
from jax import lax
from jax.experimental import pallas as pl
from jax.experimental.pallas import tpu as pltpu
from jax.experimental.pallas import tpu_sc as plsc

N = 10000
NP = 10240
G = 64
PAD = NP - 1
RB = 256
NB = NP // RB
RPT = NP // 16
CHUNK = 128


def _sc_mesh():
    return plsc.VectorSubcoreMesh(core_axis_name="c", subcore_axis_name="s")


def _make_degree_kernel(K):
    @functools.partial(
        pl.kernel,
        out_type=jax.ShapeDtypeStruct((2, NP), jnp.float32),
        mesh=_sc_mesh(),
        scratch_types=[
            pltpu.VMEM((K, CHUNK), jnp.int32),
            pltpu.VMEM((CHUNK,), jnp.float32),
            pltpu.VMEM_SHARED((NP,), jnp.float32),
        ],
    )
    def deg_kernel(dst_hbm, ones_hbm, zeros_hbm, out_hbm, dstv, onesv, deg_sh):
        c = lax.axis_index("c")
        s = lax.axis_index("s")
        pltpu.sync_copy(dst_hbm.at[c, s], dstv)
        pltpu.sync_copy(ones_hbm, onesv)
        pltpu.sync_copy(zeros_hbm, deg_sh.at[pl.ds(s * RPT, RPT)])
        plsc.subcore_barrier()

        def body(j, carry):
            pltpu.sync_copy(onesv, deg_sh.at[dstv.at[j]], add=True)
            return carry

        lax.fori_loop(0, K, body, 0)
        plsc.subcore_barrier()
        pltpu.sync_copy(deg_sh.at[pl.ds(s * RPT, RPT)],
                        out_hbm.at[c, pl.ds(s * RPT, RPT)])

    return deg_kernel


def _make_scatter_kernel(K):
    @functools.partial(
        pl.kernel,
        out_type=jax.ShapeDtypeStruct((2, NP, 128), jnp.float32),
        mesh=_sc_mesh(),
        scratch_types=[
            pltpu.VMEM((K, CHUNK), jnp.int32),
            pltpu.VMEM((K, CHUNK), jnp.int32),
            pltpu.VMEM((CHUNK, 128), jnp.float32),
            pltpu.VMEM_SHARED((NP, 128), jnp.float32),
        ],
    )
    def scat_kernel(table_hbm, src_hbm, dst_hbm, zeros_hbm, out_hbm,
                    srcv, dstv, gbuf, acc_sh):
        c = lax.axis_index("c")
        s = lax.axis_index("s")
        pltpu.sync_copy(src_hbm.at[c, s], srcv)
        pltpu.sync_copy(dst_hbm.at[c, s], dstv)
        pltpu.sync_copy(zeros_hbm, acc_sh.at[pl.ds(s * RPT, RPT)])
        plsc.subcore_barrier()

        def body(j, carry):
            pltpu.sync_copy(table_hbm.at[srcv.at[j]], gbuf)
            pltpu.sync_copy(gbuf, acc_sh.at[dstv.at[j]], add=True)
            return carry

        lax.fori_loop(0, K, body, 0)
        plsc.subcore_barrier()
        pltpu.sync_copy(acc_sh.at[pl.ds(s * RPT, RPT)],
                        out_hbm.at[c, pl.ds(s * RPT, RPT)])

    return scat_kernel


def _dis_of(deg_ref):
    deg = deg_ref[0] + deg_ref[1]
    return lax.rsqrt(jnp.maximum(deg + 1.0, 1.0))


def _scale_kernel(x_ref, deg_ref, o_ref):
    o_ref[...] = x_ref[...] * _dis_of(deg_ref)


def _tc_scale(xp, deg):
    return pl.pallas_call(
        _scale_kernel,
        grid=(NB,),
        in_specs=[
            pl.BlockSpec((RB, 128), lambda i: (i, 0)),
            pl.BlockSpec((2, RB, 1), lambda i: (0, i, 0)),
        ],
        out_specs=pl.BlockSpec((RB, 128), lambda i: (i, 0)),
        out_shape=jax.ShapeDtypeStruct((NP, 128), jnp.float32),
    )(xp, deg)


def _layer1_kernel(acc_ref, xs_ref, deg_ref, w_ref, b_ref, o_ref):
    dis = _dis_of(deg_ref)
    m = dis * (acc_ref[0] + acc_ref[1] + xs_ref[...])
    h = jnp.dot(m, w_ref[...], preferred_element_type=jnp.float32)
    h = jnp.maximum(h + b_ref[0:1, :], 0.0)
    hs = dis * h
    o_ref[0] = hs[:, :128]
    o_ref[1] = hs[:, 128:]


def _tc_layer1(acc, xs, deg, W, bt):
    return pl.pallas_call(
        _layer1_kernel,
        grid=(NB,),
        in_specs=[
            pl.BlockSpec((2, RB, 128), lambda i: (0, i, 0)),
            pl.BlockSpec((RB, 128), lambda i: (i, 0)),
            pl.BlockSpec((2, RB, 1), lambda i: (0, i, 0)),
            pl.BlockSpec((128, 256), lambda i: (0, 0)),
            pl.BlockSpec((8, 256), lambda i: (0, 0)),
        ],
        out_specs=pl.BlockSpec((2, RB, 128), lambda i: (0, i, 0)),
        out_shape=jax.ShapeDtypeStruct((2, NP, 128), jnp.float32),
    )(acc, xs, deg, W, bt)


def _layer2_kernel(acc_ref, tbl_ref, deg_ref, w_ref, b_ref, o_ref):
    dis = _dis_of(deg_ref)
    m = jnp.concatenate(
        [dis * (acc_ref[0] + tbl_ref[0]), dis * (acc_ref[1] + tbl_ref[1])],
        axis=1)
    h = jnp.dot(m, w_ref[...], preferred_element_type=jnp.float32)
    h = jnp.maximum(h + b_ref[0:1, :], 0.0)
    hs = dis * h
    o_ref[0] = hs[:, :128]
    o_ref[1] = hs[:, 128:]


def _tc_layer2(acc, tbl, deg, W, bt):
    return pl.pallas_call(
        _layer2_kernel,
        grid=(NB,),
        in_specs=[
            pl.BlockSpec((2, RB, 128), lambda i: (0, i, 0)),
            pl.BlockSpec((2, RB, 128), lambda i: (0, i, 0)),
            pl.BlockSpec((2, RB, 1), lambda i: (0, i, 0)),
            pl.BlockSpec((256, 256), lambda i: (0, 0)),
            pl.BlockSpec((8, 256), lambda i: (0, 0)),
        ],
        out_specs=pl.BlockSpec((2, RB, 128), lambda i: (0, i, 0)),
        out_shape=jax.ShapeDtypeStruct((2, NP, 128), jnp.float32),
    )(acc, tbl, deg, W, bt)


def _final_kernel(acc_ref, tbl_ref, deg_ref, w3_ref, b3_ref, wl_ref, bl_ref,
                  bt_ref, o_ref, sums, cnts):
    i = pl.program_id(0)

    @pl.when(i == 0)
    def _init():
        sums[...] = jnp.zeros_like(sums)
        cnts[...] = jnp.zeros_like(cnts)

    dis = _dis_of(deg_ref)
    m = jnp.concatenate(
        [dis * (acc_ref[0] + tbl_ref[0]), dis * (acc_ref[1] + tbl_ref[1])],
        axis=1)
    h3 = jnp.dot(m, w3_ref[...], preferred_element_type=jnp.float32)
    h3 = h3 + b3_ref[0:1, :]

    bb = jnp.broadcast_to(bt_ref[...], (RB, G))
    gi = lax.broadcasted_iota(jnp.int32, (RB, G), 1)
    oh = (bb == gi).astype(jnp.float32)
    sums[...] += lax.dot_general(oh, h3, (((0,), (0,)), ((), ())),
                                 preferred_element_type=jnp.float32)
    ones = jnp.ones((RB, 128), jnp.float32)
    cnts[...] += lax.dot_general(oh, ones, (((0,), (0,)), ((), ())),
                                 preferred_element_type=jnp.float32)

    @pl.when(i == NB - 1)
    def _fin():
        pooled = sums[...] / jnp.maximum(cnts[...][:, 0:1], 1.0)
        o_ref[...] = (jnp.dot(pooled, wl_ref[...],
                              preferred_element_type=jnp.float32)
                      + bl_ref[0:1, :])


def _tc_final(acc, tbl, deg, W3, b3t, Wlp, blt, batchp):
    return pl.pallas_call(
        _final_kernel,
        grid=(NB,),
        in_specs=[
            pl.BlockSpec((2, RB, 128), lambda i: (0, i, 0)),
            pl.BlockSpec((2, RB, 128), lambda i: (0, i, 0)),
            pl.BlockSpec((2, RB, 1), lambda i: (0, i, 0)),
            pl.BlockSpec((256, 256), lambda i: (0, 0)),
            pl.BlockSpec((8, 256), lambda i: (0, 0)),
            pl.BlockSpec((256, 128), lambda i: (0, 0)),
            pl.BlockSpec((8, 128), lambda i: (0, 0)),
            pl.BlockSpec((RB, 1), lambda i: (i, 0)),
        ],
        out_specs=pl.BlockSpec((G, 128), lambda i: (0, 0)),
        out_shape=jax.ShapeDtypeStruct((G, 128), jnp.float32),
        scratch_shapes=[
            pltpu.VMEM((G, 256), jnp.float32),
            pltpu.VMEM((G, 128), jnp.float32),
        ],
    )(acc, tbl, deg, W3, b3t, Wlp, blt, batchp)


def _pad_edges(a, total, fill):
    return jnp.concatenate(
        [a, jnp.full((total - a.shape[0],), fill, jnp.int32)])


def kernel(x, edge_index, batch, W1, b1, W2, b2, W3, b3, Wl, bl):
    E = edge_index.shape[1]
    OUT = Wl.shape[1]
    src = edge_index[0]
    dst = edge_index[1]

    K1 = -(-E // (32 * CHUNK))
    s1 = _pad_edges(src, 32 * K1 * CHUNK, PAD).reshape(2, 16, K1, CHUNK)
    d1 = _pad_edges(dst, 32 * K1 * CHUNK, PAD).reshape(2, 16, K1, CHUNK)

    K2 = -(-E // (16 * CHUNK))
    sp = _pad_edges(src, 16 * K2 * CHUNK, PAD).reshape(16, K2, CHUNK)
    dp = _pad_edges(dst, 16 * K2 * CHUNK, PAD).reshape(16, K2, CHUNK)
    s23 = jnp.stack([sp, sp + NP])
    d23 = jnp.stack([dp, dp])

    xp = jnp.pad(x, ((0, NP - N), (0, 0)))
    batchp = _pad_edges(batch, NP, G)[:, None]
    ones_c = jnp.ones((CHUNK,), jnp.float32)
    zeros_r = jnp.zeros((RPT,), jnp.float32)
    zeros_m = jnp.zeros((RPT, 128), jnp.float32)
    b1t = jnp.tile(b1[None, :], (8, 1))
    b2t = jnp.tile(b2[None, :], (8, 1))
    b3t = jnp.tile(b3[None, :], (8, 1))
    Wlp = jnp.pad(Wl, ((0, 0), (0, 128 - OUT)))
    blt = jnp.tile(jnp.pad(bl, (0, 128 - OUT))[None, :], (8, 1))

    deg = _make_degree_kernel(K1)(d1, ones_c, zeros_r)[:, :, None]

    xs = _tc_scale(xp, deg)
    acc1 = _make_scatter_kernel(K1)(xs, s1, d1, zeros_m)
    h1s = _tc_layer1(acc1, xs, deg, W1, b1t)

    scat23 = _make_scatter_kernel(K2)
    acc2 = scat23(h1s.reshape(2 * NP, 128), s23, d23, zeros_m)
    h2s = _tc_layer2(acc2, h1s, deg, W2, b2t)
    acc3 = scat23(h2s.reshape(2 * NP, 128), s23, d23, zeros_m)

    logits = _tc_final(acc3, h2s, deg, W3, b3t, Wlp, blt, batchp)
    return logits[:, :OUT]


# speedup vs baseline: 7.2377x; 7.2377x over previous
import functools

import jax
import jax.numpy as jnp
from jax import lax
from jax.experimental import pallas as pl
from jax.experimental.pallas import tpu as pltpu
from jax.experimental.pallas import tpu_sc as plsc

N = 10000
NP = 10240
G = 64
PAD = NP - 1
RB = 256
NB = NP // RB
RPT = NP // 16
CHUNK = 128
KB = 16


def _sc_mesh():
    return plsc.VectorSubcoreMesh(core_axis_name="c", subcore_axis_name="s")


def _make_degree_kernel(K):
    @functools.partial(
        pl.kernel,
        out_type=jax.ShapeDtypeStruct((2, NP), jnp.float32),
        mesh=_sc_mesh(),
        scratch_types=[
            pltpu.VMEM((K, CHUNK), jnp.int32),
            pltpu.VMEM((CHUNK,), jnp.float32),
            pltpu.VMEM_SHARED((NP,), jnp.float32),
        ],
    )
    def deg_kernel(dst_hbm, ones_hbm, zeros_hbm, out_hbm, dstv, onesv, deg_sh):
        c = lax.axis_index("c")
        s = lax.axis_index("s")
        pltpu.sync_copy(dst_hbm.at[c, s], dstv)
        pltpu.sync_copy(ones_hbm, onesv)
        pltpu.sync_copy(zeros_hbm, deg_sh.at[pl.ds(s * RPT, RPT)])
        plsc.subcore_barrier()

        def body(j, carry):
            pltpu.sync_copy(onesv, deg_sh.at[dstv.at[j]], add=True)
            return carry

        lax.fori_loop(0, K, body, 0)
        plsc.subcore_barrier()
        pltpu.sync_copy(deg_sh.at[pl.ds(s * RPT, RPT)],
                        out_hbm.at[c, pl.ds(s * RPT, RPT)])

    return deg_kernel


def _make_scatter_kernel(K):
    assert K % KB == 0

    @functools.partial(
        pl.kernel,
        out_type=jax.ShapeDtypeStruct((2, NP, 128), jnp.float32),
        mesh=_sc_mesh(),
        scratch_types=[
            pltpu.VMEM((KB, CHUNK), jnp.int32),
            pltpu.VMEM((KB, CHUNK), jnp.int32),
            pltpu.VMEM((CHUNK, 128), jnp.float32),
            pltpu.VMEM_SHARED((NP, 128), jnp.float32),
        ],
    )
    def scat_kernel(table_hbm, src_hbm, dst_hbm, zeros_hbm, out_hbm,
                    srcv, dstv, gbuf, acc_sh):
        c = lax.axis_index("c")
        s = lax.axis_index("s")
        pltpu.sync_copy(zeros_hbm, acc_sh.at[pl.ds(s * RPT, RPT)])
        plsc.subcore_barrier()

        def outer(b, carry):
            pltpu.sync_copy(src_hbm.at[c, s, pl.ds(b * KB, KB)], srcv)
            pltpu.sync_copy(dst_hbm.at[c, s, pl.ds(b * KB, KB)], dstv)

            def body(j, cc):
                pltpu.sync_copy(table_hbm.at[srcv.at[j]], gbuf)
                pltpu.sync_copy(gbuf, acc_sh.at[dstv.at[j]], add=True)
                return cc

            lax.fori_loop(0, KB, body, 0)
            return carry

        lax.fori_loop(0, K // KB, outer, 0)
        plsc.subcore_barrier()
        pltpu.sync_copy(acc_sh.at[pl.ds(s * RPT, RPT)],
                        out_hbm.at[c, pl.ds(s * RPT, RPT)])

    return scat_kernel


def _dis_of(deg_ref):
    deg = deg_ref[0] + deg_ref[1]
    return lax.rsqrt(jnp.maximum(deg + 1.0, 1.0))


def _scale_kernel(x_ref, deg_ref, o_ref):
    o_ref[...] = x_ref[...] * _dis_of(deg_ref)


def _tc_scale(xp, deg):
    return pl.pallas_call(
        _scale_kernel,
        grid=(NB,),
        in_specs=[
            pl.BlockSpec((RB, 128), lambda i: (i, 0)),
            pl.BlockSpec((2, RB, 1), lambda i: (0, i, 0)),
        ],
        out_specs=pl.BlockSpec((RB, 128), lambda i: (i, 0)),
        out_shape=jax.ShapeDtypeStruct((NP, 128), jnp.float32),
    )(xp, deg)


def _layer1_kernel(acc_ref, xs_ref, deg_ref, w_ref, b_ref, o_ref):
    dis = _dis_of(deg_ref)
    m = dis * (acc_ref[0] + acc_ref[1] + xs_ref[...])
    h = jnp.dot(m, w_ref[...], preferred_element_type=jnp.float32)
    h = jnp.maximum(h + b_ref[0:1, :], 0.0)
    hs = dis * h
    o_ref[0] = hs[:, :128]
    o_ref[1] = hs[:, 128:]


def _tc_layer1(acc, xs, deg, W, bt):
    return pl.pallas_call(
        _layer1_kernel,
        grid=(NB,),
        in_specs=[
            pl.BlockSpec((2, RB, 128), lambda i: (0, i, 0)),
            pl.BlockSpec((RB, 128), lambda i: (i, 0)),
            pl.BlockSpec((2, RB, 1), lambda i: (0, i, 0)),
            pl.BlockSpec((128, 256), lambda i: (0, 0)),
            pl.BlockSpec((8, 256), lambda i: (0, 0)),
        ],
        out_specs=pl.BlockSpec((2, RB, 128), lambda i: (0, i, 0)),
        out_shape=jax.ShapeDtypeStruct((2, NP, 128), jnp.float32),
    )(acc, xs, deg, W, bt)


def _layer2_kernel(acc_ref, tbl_ref, deg_ref, w_ref, b_ref, o_ref):
    dis = _dis_of(deg_ref)
    m = jnp.concatenate(
        [dis * (acc_ref[0] + tbl_ref[0]), dis * (acc_ref[1] + tbl_ref[1])],
        axis=1)
    h = jnp.dot(m, w_ref[...], preferred_element_type=jnp.float32)
    h = jnp.maximum(h + b_ref[0:1, :], 0.0)
    hs = dis * h
    o_ref[0] = hs[:, :128]
    o_ref[1] = hs[:, 128:]


def _tc_layer2(acc, tbl, deg, W, bt):
    return pl.pallas_call(
        _layer2_kernel,
        grid=(NB,),
        in_specs=[
            pl.BlockSpec((2, RB, 128), lambda i: (0, i, 0)),
            pl.BlockSpec((2, RB, 128), lambda i: (0, i, 0)),
            pl.BlockSpec((2, RB, 1), lambda i: (0, i, 0)),
            pl.BlockSpec((256, 256), lambda i: (0, 0)),
            pl.BlockSpec((8, 256), lambda i: (0, 0)),
        ],
        out_specs=pl.BlockSpec((2, RB, 128), lambda i: (0, i, 0)),
        out_shape=jax.ShapeDtypeStruct((2, NP, 128), jnp.float32),
    )(acc, tbl, deg, W, bt)


def _final_kernel(acc_ref, tbl_ref, deg_ref, w3_ref, b3_ref, wl_ref, bl_ref,
                  bt_ref, o_ref, sums, cnts):
    i = pl.program_id(0)

    @pl.when(i == 0)
    def _init():
        sums[...] = jnp.zeros_like(sums)
        cnts[...] = jnp.zeros_like(cnts)

    dis = _dis_of(deg_ref)
    m = jnp.concatenate(
        [dis * (acc_ref[0] + tbl_ref[0]), dis * (acc_ref[1] + tbl_ref[1])],
        axis=1)
    h3 = jnp.dot(m, w3_ref[...], preferred_element_type=jnp.float32)
    h3 = h3 + b3_ref[0:1, :]

    bb = jnp.broadcast_to(bt_ref[...], (RB, G))
    gi = lax.broadcasted_iota(jnp.int32, (RB, G), 1)
    oh = (bb == gi).astype(jnp.float32)
    sums[...] += lax.dot_general(oh, h3, (((0,), (0,)), ((), ())),
                                 preferred_element_type=jnp.float32)
    ones = jnp.ones((RB, 128), jnp.float32)
    cnts[...] += lax.dot_general(oh, ones, (((0,), (0,)), ((), ())),
                                 preferred_element_type=jnp.float32)

    @pl.when(i == NB - 1)
    def _fin():
        pooled = sums[...] / jnp.maximum(cnts[...][:, 0:1], 1.0)
        o_ref[...] = (jnp.dot(pooled, wl_ref[...],
                              preferred_element_type=jnp.float32)
                      + bl_ref[0:1, :])


def _tc_final(acc, tbl, deg, W3, b3t, Wlp, blt, batchp):
    return pl.pallas_call(
        _final_kernel,
        grid=(NB,),
        in_specs=[
            pl.BlockSpec((2, RB, 128), lambda i: (0, i, 0)),
            pl.BlockSpec((2, RB, 128), lambda i: (0, i, 0)),
            pl.BlockSpec((2, RB, 1), lambda i: (0, i, 0)),
            pl.BlockSpec((256, 256), lambda i: (0, 0)),
            pl.BlockSpec((8, 256), lambda i: (0, 0)),
            pl.BlockSpec((256, 128), lambda i: (0, 0)),
            pl.BlockSpec((8, 128), lambda i: (0, 0)),
            pl.BlockSpec((RB, 1), lambda i: (i, 0)),
        ],
        out_specs=pl.BlockSpec((G, 128), lambda i: (0, 0)),
        out_shape=jax.ShapeDtypeStruct((G, 128), jnp.float32),
        scratch_shapes=[
            pltpu.VMEM((G, 256), jnp.float32),
            pltpu.VMEM((G, 128), jnp.float32),
        ],
    )(acc, tbl, deg, W3, b3t, Wlp, blt, batchp)


def _pad_edges(a, total, fill):
    return jnp.concatenate(
        [a, jnp.full((total - a.shape[0],), fill, jnp.int32)])


def kernel(x, edge_index, batch, W1, b1, W2, b2, W3, b3, Wl, bl):
    E = edge_index.shape[1]
    OUT = Wl.shape[1]
    src = edge_index[0]
    dst = edge_index[1]

    def _ceil_mult(a, b):
        return -(-a // b) * b

    K1 = _ceil_mult(-(-E // (32 * CHUNK)), KB)
    s1 = _pad_edges(src, 32 * K1 * CHUNK, PAD).reshape(2, 16, K1, CHUNK)
    d1 = _pad_edges(dst, 32 * K1 * CHUNK, PAD).reshape(2, 16, K1, CHUNK)

    K2 = _ceil_mult(-(-E // (16 * CHUNK)), KB)
    sp = _pad_edges(src, 16 * K2 * CHUNK, PAD).reshape(16, K2, CHUNK)
    dp = _pad_edges(dst, 16 * K2 * CHUNK, PAD).reshape(16, K2, CHUNK)
    s23 = jnp.stack([sp, sp + NP])
    d23 = jnp.stack([dp, dp])

    xp = jnp.pad(x, ((0, NP - N), (0, 0)))
    batchp = _pad_edges(batch, NP, G)[:, None]
    ones_c = jnp.ones((CHUNK,), jnp.float32)
    zeros_r = jnp.zeros((RPT,), jnp.float32)
    zeros_m = jnp.zeros((RPT, 128), jnp.float32)
    b1t = jnp.tile(b1[None, :], (8, 1))
    b2t = jnp.tile(b2[None, :], (8, 1))
    b3t = jnp.tile(b3[None, :], (8, 1))
    Wlp = jnp.pad(Wl, ((0, 0), (0, 128 - OUT)))
    blt = jnp.tile(jnp.pad(bl, (0, 128 - OUT))[None, :], (8, 1))

    deg = _make_degree_kernel(K1)(d1, ones_c, zeros_r)[:, :, None]

    xs = _tc_scale(xp, deg)
    acc1 = _make_scatter_kernel(K1)(xs, s1, d1, zeros_m)
    h1s = _tc_layer1(acc1, xs, deg, W1, b1t)

    scat23 = _make_scatter_kernel(K2)
    acc2 = scat23(h1s.reshape(2 * NP, 128), s23, d23, zeros_m)
    h2s = _tc_layer2(acc2, h1s, deg, W2, b2t)
    acc3 = scat23(h2s.reshape(2 * NP, 128), s23, d23, zeros_m)

    logits = _tc_final(acc3, h2s, deg, W3, b3t, Wlp, blt, batchp)
    return logits[:, :OUT]


# speedup vs baseline: 8.0511x; 1.1124x over previous
import functools

import jax
import jax.numpy as jnp
from jax import lax
from jax.experimental import pallas as pl
from jax.experimental.pallas import tpu as pltpu
from jax.experimental.pallas import tpu_sc as plsc

N = 10000
NP = 10240
G = 64
PAD = NP - 1
RB = 256
NB = NP // RB
RPT = NP // 16
CHUNK = 128
KB = 16


def _sc_mesh():
    return plsc.VectorSubcoreMesh(core_axis_name="c", subcore_axis_name="s")


def _make_degree_kernel(K):
    @functools.partial(
        pl.kernel,
        out_type=jax.ShapeDtypeStruct((2, NP), jnp.float32),
        mesh=_sc_mesh(),
        scratch_types=[
            pltpu.VMEM((K, CHUNK), jnp.int32),
            pltpu.VMEM((CHUNK,), jnp.float32),
            pltpu.VMEM_SHARED((NP,), jnp.float32),
        ],
    )
    def deg_kernel(dst_hbm, ones_hbm, zeros_hbm, out_hbm, dstv, onesv, deg_sh):
        c = lax.axis_index("c")
        s = lax.axis_index("s")
        pltpu.sync_copy(dst_hbm.at[c, s], dstv)
        pltpu.sync_copy(ones_hbm, onesv)
        pltpu.sync_copy(zeros_hbm, deg_sh.at[pl.ds(s * RPT, RPT)])
        plsc.subcore_barrier()

        def body(j, carry):
            pltpu.sync_copy(onesv, deg_sh.at[dstv.at[j]], add=True)
            return carry

        lax.fori_loop(0, K, body, 0)
        plsc.subcore_barrier()
        pltpu.sync_copy(deg_sh.at[pl.ds(s * RPT, RPT)],
                        out_hbm.at[c, pl.ds(s * RPT, RPT)])

    return deg_kernel


def _make_scatter_kernel(K):
    assert K % KB == 0

    @functools.partial(
        pl.kernel,
        out_type=jax.ShapeDtypeStruct((2, NP, 128), jnp.float32),
        mesh=_sc_mesh(),
        scratch_types=[
            pltpu.VMEM((KB, CHUNK), jnp.int32),
            pltpu.VMEM((KB, CHUNK), jnp.int32),
            pltpu.VMEM((CHUNK, 128), jnp.float32),
            pltpu.VMEM((CHUNK, 128), jnp.float32),
            pltpu.VMEM_SHARED((NP, 128), jnp.float32),
            pltpu.SemaphoreType.DMA,
            pltpu.SemaphoreType.DMA,
        ],
    )
    def scat_kernel(table_hbm, src_hbm, dst_hbm, zeros_hbm, out_hbm,
                    srcv, dstv, gbuf0, gbuf1, acc_sh, sem0, sem1):
        c = lax.axis_index("c")
        s = lax.axis_index("s")
        bufs = (gbuf0, gbuf1)
        sems = (sem0, sem1)
        pltpu.sync_copy(zeros_hbm, acc_sh.at[pl.ds(s * RPT, RPT)])
        plsc.subcore_barrier()

        def outer(b, carry):
            pltpu.sync_copy(src_hbm.at[c, s, pl.ds(b * KB, KB)], srcv)
            pltpu.sync_copy(dst_hbm.at[c, s, pl.ds(b * KB, KB)], dstv)
            pending = pltpu.async_copy(table_hbm.at[srcv.at[0]], bufs[0],
                                       sems[0])
            for j in range(KB):
                pending.wait()
                if j + 1 < KB:
                    pending = pltpu.async_copy(
                        table_hbm.at[srcv.at[j + 1]], bufs[(j + 1) % 2],
                        sems[(j + 1) % 2])
                pltpu.sync_copy(bufs[j % 2], acc_sh.at[dstv.at[j]], add=True)
            return carry

        lax.fori_loop(0, K // KB, outer, 0)
        plsc.subcore_barrier()
        pltpu.sync_copy(acc_sh.at[pl.ds(s * RPT, RPT)],
                        out_hbm.at[c, pl.ds(s * RPT, RPT)])

    return scat_kernel


def _dis_of(deg_ref):
    deg = deg_ref[0] + deg_ref[1]
    return lax.rsqrt(jnp.maximum(deg + 1.0, 1.0))


def _scale_kernel(x_ref, deg_ref, o_ref):
    o_ref[...] = x_ref[...] * _dis_of(deg_ref)


def _tc_scale(xp, deg):
    return pl.pallas_call(
        _scale_kernel,
        grid=(NB,),
        in_specs=[
            pl.BlockSpec((RB, 128), lambda i: (i, 0)),
            pl.BlockSpec((2, RB, 1), lambda i: (0, i, 0)),
        ],
        out_specs=pl.BlockSpec((RB, 128), lambda i: (i, 0)),
        out_shape=jax.ShapeDtypeStruct((NP, 128), jnp.float32),
    )(xp, deg)


def _layer1_kernel(acc_ref, xs_ref, deg_ref, w_ref, b_ref, o_ref):
    dis = _dis_of(deg_ref)
    m = dis * (acc_ref[0] + acc_ref[1] + xs_ref[...])
    h = jnp.dot(m, w_ref[...], preferred_element_type=jnp.float32)
    h = jnp.maximum(h + b_ref[0:1, :], 0.0)
    hs = dis * h
    o_ref[0] = hs[:, :128]
    o_ref[1] = hs[:, 128:]


def _tc_layer1(acc, xs, deg, W, bt):
    return pl.pallas_call(
        _layer1_kernel,
        grid=(NB,),
        in_specs=[
            pl.BlockSpec((2, RB, 128), lambda i: (0, i, 0)),
            pl.BlockSpec((RB, 128), lambda i: (i, 0)),
            pl.BlockSpec((2, RB, 1), lambda i: (0, i, 0)),
            pl.BlockSpec((128, 256), lambda i: (0, 0)),
            pl.BlockSpec((8, 256), lambda i: (0, 0)),
        ],
        out_specs=pl.BlockSpec((2, RB, 128), lambda i: (0, i, 0)),
        out_shape=jax.ShapeDtypeStruct((2, NP, 128), jnp.float32),
    )(acc, xs, deg, W, bt)


def _layer2_kernel(acc_ref, tbl_ref, deg_ref, w_ref, b_ref, o_ref):
    dis = _dis_of(deg_ref)
    m = jnp.concatenate(
        [dis * (acc_ref[0] + tbl_ref[0]), dis * (acc_ref[1] + tbl_ref[1])],
        axis=1)
    h = jnp.dot(m, w_ref[...], preferred_element_type=jnp.float32)
    h = jnp.maximum(h + b_ref[0:1, :], 0.0)
    hs = dis * h
    o_ref[0] = hs[:, :128]
    o_ref[1] = hs[:, 128:]


def _tc_layer2(acc, tbl, deg, W, bt):
    return pl.pallas_call(
        _layer2_kernel,
        grid=(NB,),
        in_specs=[
            pl.BlockSpec((2, RB, 128), lambda i: (0, i, 0)),
            pl.BlockSpec((2, RB, 128), lambda i: (0, i, 0)),
            pl.BlockSpec((2, RB, 1), lambda i: (0, i, 0)),
            pl.BlockSpec((256, 256), lambda i: (0, 0)),
            pl.BlockSpec((8, 256), lambda i: (0, 0)),
        ],
        out_specs=pl.BlockSpec((2, RB, 128), lambda i: (0, i, 0)),
        out_shape=jax.ShapeDtypeStruct((2, NP, 128), jnp.float32),
    )(acc, tbl, deg, W, bt)


def _final_kernel(acc_ref, tbl_ref, deg_ref, w3_ref, b3_ref, wl_ref, bl_ref,
                  bt_ref, o_ref, sums, cnts):
    i = pl.program_id(0)

    @pl.when(i == 0)
    def _init():
        sums[...] = jnp.zeros_like(sums)
        cnts[...] = jnp.zeros_like(cnts)

    dis = _dis_of(deg_ref)
    m = jnp.concatenate(
        [dis * (acc_ref[0] + tbl_ref[0]), dis * (acc_ref[1] + tbl_ref[1])],
        axis=1)
    h3 = jnp.dot(m, w3_ref[...], preferred_element_type=jnp.float32)
    h3 = h3 + b3_ref[0:1, :]

    bb = jnp.broadcast_to(bt_ref[...], (RB, G))
    gi = lax.broadcasted_iota(jnp.int32, (RB, G), 1)
    oh = (bb == gi).astype(jnp.float32)
    sums[...] += lax.dot_general(oh, h3, (((0,), (0,)), ((), ())),
                                 preferred_element_type=jnp.float32)
    ones = jnp.ones((RB, 128), jnp.float32)
    cnts[...] += lax.dot_general(oh, ones, (((0,), (0,)), ((), ())),
                                 preferred_element_type=jnp.float32)

    @pl.when(i == NB - 1)
    def _fin():
        pooled = sums[...] / jnp.maximum(cnts[...][:, 0:1], 1.0)
        o_ref[...] = (jnp.dot(pooled, wl_ref[...],
                              preferred_element_type=jnp.float32)
                      + bl_ref[0:1, :])


def _tc_final(acc, tbl, deg, W3, b3t, Wlp, blt, batchp):
    return pl.pallas_call(
        _final_kernel,
        grid=(NB,),
        in_specs=[
            pl.BlockSpec((2, RB, 128), lambda i: (0, i, 0)),
            pl.BlockSpec((2, RB, 128), lambda i: (0, i, 0)),
            pl.BlockSpec((2, RB, 1), lambda i: (0, i, 0)),
            pl.BlockSpec((256, 256), lambda i: (0, 0)),
            pl.BlockSpec((8, 256), lambda i: (0, 0)),
            pl.BlockSpec((256, 128), lambda i: (0, 0)),
            pl.BlockSpec((8, 128), lambda i: (0, 0)),
            pl.BlockSpec((RB, 1), lambda i: (i, 0)),
        ],
        out_specs=pl.BlockSpec((G, 128), lambda i: (0, 0)),
        out_shape=jax.ShapeDtypeStruct((G, 128), jnp.float32),
        scratch_shapes=[
            pltpu.VMEM((G, 256), jnp.float32),
            pltpu.VMEM((G, 128), jnp.float32),
        ],
    )(acc, tbl, deg, W3, b3t, Wlp, blt, batchp)


def _pad_edges(a, total, fill):
    return jnp.concatenate(
        [a, jnp.full((total - a.shape[0],), fill, jnp.int32)])


def kernel(x, edge_index, batch, W1, b1, W2, b2, W3, b3, Wl, bl):
    E = edge_index.shape[1]
    OUT = Wl.shape[1]
    src = edge_index[0]
    dst = edge_index[1]

    def _ceil_mult(a, b):
        return -(-a // b) * b

    K1 = _ceil_mult(-(-E // (32 * CHUNK)), KB)
    s1 = _pad_edges(src, 32 * K1 * CHUNK, PAD).reshape(2, 16, K1, CHUNK)
    d1 = _pad_edges(dst, 32 * K1 * CHUNK, PAD).reshape(2, 16, K1, CHUNK)

    K2 = _ceil_mult(-(-E // (16 * CHUNK)), KB)
    sp = _pad_edges(src, 16 * K2 * CHUNK, PAD).reshape(16, K2, CHUNK)
    dp = _pad_edges(dst, 16 * K2 * CHUNK, PAD).reshape(16, K2, CHUNK)
    s23 = jnp.stack([sp, sp + NP])
    d23 = jnp.stack([dp, dp])

    xp = jnp.pad(x, ((0, NP - N), (0, 0)))
    batchp = _pad_edges(batch, NP, G)[:, None]
    ones_c = jnp.ones((CHUNK,), jnp.float32)
    zeros_r = jnp.zeros((RPT,), jnp.float32)
    zeros_m = jnp.zeros((RPT, 128), jnp.float32)
    b1t = jnp.tile(b1[None, :], (8, 1))
    b2t = jnp.tile(b2[None, :], (8, 1))
    b3t = jnp.tile(b3[None, :], (8, 1))
    Wlp = jnp.pad(Wl, ((0, 0), (0, 128 - OUT)))
    blt = jnp.tile(jnp.pad(bl, (0, 128 - OUT))[None, :], (8, 1))

    deg = _make_degree_kernel(K1)(d1, ones_c, zeros_r)[:, :, None]

    xs = _tc_scale(xp, deg)
    acc1 = _make_scatter_kernel(K1)(xs, s1, d1, zeros_m)
    h1s = _tc_layer1(acc1, xs, deg, W1, b1t)

    scat23 = _make_scatter_kernel(K2)
    acc2 = scat23(h1s.reshape(2 * NP, 128), s23, d23, zeros_m)
    h2s = _tc_layer2(acc2, h1s, deg, W2, b2t)
    acc3 = scat23(h2s.reshape(2 * NP, 128), s23, d23, zeros_m)

    logits = _tc_final(acc3, h2s, deg, W3, b3t, Wlp, blt, batchp)
    return logits[:, :OUT]


# speedup vs baseline: 8.4552x; 1.0502x over previous
import functools

import jax
import jax.numpy as jnp
from jax import lax
from jax.experimental import pallas as pl
from jax.experimental.pallas import tpu as pltpu
from jax.experimental.pallas import tpu_sc as plsc

N = 10000
NP = 10240
G = 64
PAD = NP - 1
RB = 256
NB = NP // RB
RPT = NP // 16
CHUNK = 128
KB = 16


def _sc_mesh():
    return plsc.VectorSubcoreMesh(core_axis_name="c", subcore_axis_name="s")


def _make_degree_kernel(K):
    @functools.partial(
        pl.kernel,
        out_type=jax.ShapeDtypeStruct((2, NP), jnp.float32),
        mesh=_sc_mesh(),
        scratch_types=[
            pltpu.VMEM((K, CHUNK), jnp.int32),
            pltpu.VMEM((CHUNK,), jnp.float32),
            pltpu.VMEM_SHARED((NP,), jnp.float32),
        ],
    )
    def deg_kernel(dst_hbm, ones_hbm, zeros_hbm, out_hbm, dstv, onesv, deg_sh):
        c = lax.axis_index("c")
        s = lax.axis_index("s")
        pltpu.sync_copy(dst_hbm.at[c, s], dstv)
        pltpu.sync_copy(ones_hbm, onesv)
        pltpu.sync_copy(zeros_hbm, deg_sh.at[pl.ds(s * RPT, RPT)])
        plsc.subcore_barrier()

        def body(j, carry):
            pltpu.sync_copy(onesv, deg_sh.at[dstv.at[j]], add=True)
            return carry

        lax.fori_loop(0, K, body, 0)
        plsc.subcore_barrier()
        pltpu.sync_copy(deg_sh.at[pl.ds(s * RPT, RPT)],
                        out_hbm.at[c, pl.ds(s * RPT, RPT)])

    return deg_kernel


def _make_scatter_kernel(K):
    assert K % KB == 0

    @functools.partial(
        pl.kernel,
        out_type=jax.ShapeDtypeStruct((2, NP, 128), jnp.float32),
        mesh=_sc_mesh(),
        scratch_types=[
            pltpu.VMEM((KB, CHUNK), jnp.int32),
            pltpu.VMEM((KB, CHUNK), jnp.int32),
            pltpu.VMEM((CHUNK, 128), jnp.float32),
            pltpu.VMEM((CHUNK, 128), jnp.float32),
            pltpu.VMEM_SHARED((NP, 128), jnp.float32),
            pltpu.SemaphoreType.DMA,
            pltpu.SemaphoreType.DMA,
            pltpu.SemaphoreType.DMA,
            pltpu.SemaphoreType.DMA,
        ],
    )
    def scat_kernel(table_hbm, src_hbm, dst_hbm, zeros_hbm, out_hbm,
                    srcv, dstv, gbuf0, gbuf1, acc_sh, gs0, gs1, ss0, ss1):
        c = lax.axis_index("c")
        s = lax.axis_index("s")
        bufs = (gbuf0, gbuf1)
        gsem = (gs0, gs1)
        ssem = (ss0, ss1)
        pltpu.sync_copy(zeros_hbm, acc_sh.at[pl.ds(s * RPT, RPT)])
        plsc.subcore_barrier()

        def outer(b, carry):
            pltpu.sync_copy(src_hbm.at[c, s, pl.ds(b * KB, KB)], srcv)
            pltpu.sync_copy(dst_hbm.at[c, s, pl.ds(b * KB, KB)], dstv)
            pg = [None, None]
            pg[0] = pltpu.async_copy(table_hbm.at[srcv.at[0]], bufs[0],
                                     gsem[0])
            if KB > 1:
                pg[1] = pltpu.async_copy(table_hbm.at[srcv.at[1]], bufs[1],
                                         gsem[1])
            tail = [None, None]
            for j in range(KB):
                bj = j % 2
                pg[bj].wait()
                sd = pltpu.async_copy(bufs[bj], acc_sh.at[dstv.at[j]],
                                      ssem[bj], add=True)
                if j + 2 < KB:
                    sd.wait()
                    pg[bj] = pltpu.async_copy(table_hbm.at[srcv.at[j + 2]],
                                              bufs[bj], gsem[bj])
                else:
                    tail[bj] = sd
            for sd in tail:
                if sd is not None:
                    sd.wait()
            return carry

        lax.fori_loop(0, K // KB, outer, 0)
        plsc.subcore_barrier()
        pltpu.sync_copy(acc_sh.at[pl.ds(s * RPT, RPT)],
                        out_hbm.at[c, pl.ds(s * RPT, RPT)])

    return scat_kernel


def _dis_of(deg_ref):
    deg = deg_ref[0] + deg_ref[1]
    return lax.rsqrt(jnp.maximum(deg + 1.0, 1.0))


def _scale_kernel(x_ref, deg_ref, o_ref):
    o_ref[...] = x_ref[...] * _dis_of(deg_ref)


def _tc_scale(xp, deg):
    return pl.pallas_call(
        _scale_kernel,
        grid=(NB,),
        in_specs=[
            pl.BlockSpec((RB, 128), lambda i: (i, 0)),
            pl.BlockSpec((2, RB, 1), lambda i: (0, i, 0)),
        ],
        out_specs=pl.BlockSpec((RB, 128), lambda i: (i, 0)),
        out_shape=jax.ShapeDtypeStruct((NP, 128), jnp.float32),
    )(xp, deg)


def _layer1_kernel(acc_ref, xs_ref, deg_ref, w_ref, b_ref, o_ref):
    dis = _dis_of(deg_ref)
    m = dis * (acc_ref[0] + acc_ref[1] + xs_ref[...])
    h = jnp.dot(m, w_ref[...], preferred_element_type=jnp.float32)
    h = jnp.maximum(h + b_ref[0:1, :], 0.0)
    hs = dis * h
    o_ref[0] = hs[:, :128]
    o_ref[1] = hs[:, 128:]


def _tc_layer1(acc, xs, deg, W, bt):
    return pl.pallas_call(
        _layer1_kernel,
        grid=(NB,),
        in_specs=[
            pl.BlockSpec((2, RB, 128), lambda i: (0, i, 0)),
            pl.BlockSpec((RB, 128), lambda i: (i, 0)),
            pl.BlockSpec((2, RB, 1), lambda i: (0, i, 0)),
            pl.BlockSpec((128, 256), lambda i: (0, 0)),
            pl.BlockSpec((8, 256), lambda i: (0, 0)),
        ],
        out_specs=pl.BlockSpec((2, RB, 128), lambda i: (0, i, 0)),
        out_shape=jax.ShapeDtypeStruct((2, NP, 128), jnp.float32),
    )(acc, xs, deg, W, bt)


def _layer2_kernel(acc_ref, tbl_ref, deg_ref, w_ref, b_ref, o_ref):
    dis = _dis_of(deg_ref)
    m = jnp.concatenate(
        [dis * (acc_ref[0] + tbl_ref[0]), dis * (acc_ref[1] + tbl_ref[1])],
        axis=1)
    h = jnp.dot(m, w_ref[...], preferred_element_type=jnp.float32)
    h = jnp.maximum(h + b_ref[0:1, :], 0.0)
    hs = dis * h
    o_ref[0] = hs[:, :128]
    o_ref[1] = hs[:, 128:]


def _tc_layer2(acc, tbl, deg, W, bt):
    return pl.pallas_call(
        _layer2_kernel,
        grid=(NB,),
        in_specs=[
            pl.BlockSpec((2, RB, 128), lambda i: (0, i, 0)),
            pl.BlockSpec((2, RB, 128), lambda i: (0, i, 0)),
            pl.BlockSpec((2, RB, 1), lambda i: (0, i, 0)),
            pl.BlockSpec((256, 256), lambda i: (0, 0)),
            pl.BlockSpec((8, 256), lambda i: (0, 0)),
        ],
        out_specs=pl.BlockSpec((2, RB, 128), lambda i: (0, i, 0)),
        out_shape=jax.ShapeDtypeStruct((2, NP, 128), jnp.float32),
    )(acc, tbl, deg, W, bt)


def _final_kernel(acc_ref, tbl_ref, deg_ref, w3_ref, b3_ref, wl_ref, bl_ref,
                  bt_ref, o_ref, sums, cnts):
    i = pl.program_id(0)

    @pl.when(i == 0)
    def _init():
        sums[...] = jnp.zeros_like(sums)
        cnts[...] = jnp.zeros_like(cnts)

    dis = _dis_of(deg_ref)
    m = jnp.concatenate(
        [dis * (acc_ref[0] + tbl_ref[0]), dis * (acc_ref[1] + tbl_ref[1])],
        axis=1)
    h3 = jnp.dot(m, w3_ref[...], preferred_element_type=jnp.float32)
    h3 = h3 + b3_ref[0:1, :]

    bb = jnp.broadcast_to(bt_ref[...], (RB, G))
    gi = lax.broadcasted_iota(jnp.int32, (RB, G), 1)
    oh = (bb == gi).astype(jnp.float32)
    sums[...] += lax.dot_general(oh, h3, (((0,), (0,)), ((), ())),
                                 preferred_element_type=jnp.float32)
    ones = jnp.ones((RB, 128), jnp.float32)
    cnts[...] += lax.dot_general(oh, ones, (((0,), (0,)), ((), ())),
                                 preferred_element_type=jnp.float32)

    @pl.when(i == NB - 1)
    def _fin():
        pooled = sums[...] / jnp.maximum(cnts[...][:, 0:1], 1.0)
        o_ref[...] = (jnp.dot(pooled, wl_ref[...],
                              preferred_element_type=jnp.float32)
                      + bl_ref[0:1, :])


def _tc_final(acc, tbl, deg, W3, b3t, Wlp, blt, batchp):
    return pl.pallas_call(
        _final_kernel,
        grid=(NB,),
        in_specs=[
            pl.BlockSpec((2, RB, 128), lambda i: (0, i, 0)),
            pl.BlockSpec((2, RB, 128), lambda i: (0, i, 0)),
            pl.BlockSpec((2, RB, 1), lambda i: (0, i, 0)),
            pl.BlockSpec((256, 256), lambda i: (0, 0)),
            pl.BlockSpec((8, 256), lambda i: (0, 0)),
            pl.BlockSpec((256, 128), lambda i: (0, 0)),
            pl.BlockSpec((8, 128), lambda i: (0, 0)),
            pl.BlockSpec((RB, 1), lambda i: (i, 0)),
        ],
        out_specs=pl.BlockSpec((G, 128), lambda i: (0, 0)),
        out_shape=jax.ShapeDtypeStruct((G, 128), jnp.float32),
        scratch_shapes=[
            pltpu.VMEM((G, 256), jnp.float32),
            pltpu.VMEM((G, 128), jnp.float32),
        ],
    )(acc, tbl, deg, W3, b3t, Wlp, blt, batchp)


def _pad_edges(a, total, fill):
    return jnp.concatenate(
        [a, jnp.full((total - a.shape[0],), fill, jnp.int32)])


def kernel(x, edge_index, batch, W1, b1, W2, b2, W3, b3, Wl, bl):
    E = edge_index.shape[1]
    OUT = Wl.shape[1]
    src = edge_index[0]
    dst = edge_index[1]

    def _ceil_mult(a, b):
        return -(-a // b) * b

    K1 = _ceil_mult(-(-E // (32 * CHUNK)), KB)
    s1 = _pad_edges(src, 32 * K1 * CHUNK, PAD).reshape(2, 16, K1, CHUNK)
    d1 = _pad_edges(dst, 32 * K1 * CHUNK, PAD).reshape(2, 16, K1, CHUNK)

    K2 = _ceil_mult(-(-E // (16 * CHUNK)), KB)
    sp = _pad_edges(src, 16 * K2 * CHUNK, PAD).reshape(16, K2, CHUNK)
    dp = _pad_edges(dst, 16 * K2 * CHUNK, PAD).reshape(16, K2, CHUNK)
    s23 = jnp.stack([sp, sp + NP])
    d23 = jnp.stack([dp, dp])

    xp = jnp.pad(x, ((0, NP - N), (0, 0)))
    batchp = _pad_edges(batch, NP, G)[:, None]
    ones_c = jnp.ones((CHUNK,), jnp.float32)
    zeros_r = jnp.zeros((RPT,), jnp.float32)
    zeros_m = jnp.zeros((RPT, 128), jnp.float32)
    b1t = jnp.tile(b1[None, :], (8, 1))
    b2t = jnp.tile(b2[None, :], (8, 1))
    b3t = jnp.tile(b3[None, :], (8, 1))
    Wlp = jnp.pad(Wl, ((0, 0), (0, 128 - OUT)))
    blt = jnp.tile(jnp.pad(bl, (0, 128 - OUT))[None, :], (8, 1))

    deg = _make_degree_kernel(K1)(d1, ones_c, zeros_r)[:, :, None]

    xs = _tc_scale(xp, deg)
    acc1 = _make_scatter_kernel(K1)(xs, s1, d1, zeros_m)
    h1s = _tc_layer1(acc1, xs, deg, W1, b1t)

    scat23 = _make_scatter_kernel(K2)
    acc2 = scat23(h1s.reshape(2 * NP, 128), s23, d23, zeros_m)
    h2s = _tc_layer2(acc2, h1s, deg, W2, b2t)
    acc3 = scat23(h2s.reshape(2 * NP, 128), s23, d23, zeros_m)

    logits = _tc_final(acc3, h2s, deg, W3, b3t, Wlp, blt, batchp)
    return logits[:, :OUT]
